# Initial kernel scaffold; baseline (speedup 1.0000x reference)
import jax
import jax.numpy as jnp
from jax.experimental import pallas as pl


def kernel(h, edge_index, coord, eW1, eb1, eW2, eb2, nW1, nb1, nW2, nb2):
    raise NotImplementedError("write your pallas kernel here")


# speedup vs baseline: 1.9296x; 1.9296x over previous
import functools

import jax
import jax.numpy as jnp
from jax import lax
from jax.experimental import pallas as pl
from jax.experimental.pallas import tpu as pltpu
from jax.experimental.pallas import tpu_sc as plsc

N = 10000
E = 320000
D = 128
NC, NS = 2, 16
NW = NC * NS
PER_W = E // NW
C = 80
NCH = PER_W // C
RPT = N // NS
NCR = 1280
BN = 400
BE = 512

_f32 = jnp.float32
_i32 = jnp.int32


def _pre_body(h_ref, wa_ref, wb_ref, a_ref, b_ref):
    hb = h_ref[...]
    a_ref[...] = jnp.dot(hb, wa_ref[...], preferred_element_type=_f32)
    b_ref[...] = jnp.dot(hb, wb_ref[...], preferred_element_type=_f32)


def _precompute(h, Wa, Wb):
    return pl.pallas_call(
        _pre_body,
        grid=(N // BN,),
        in_specs=[
            pl.BlockSpec((BN, D), lambda i: (i, 0)),
            pl.BlockSpec((D, D), lambda i: (0, 0)),
            pl.BlockSpec((D, D), lambda i: (0, 0)),
        ],
        out_specs=[
            pl.BlockSpec((BN, D), lambda i: (i, 0)),
            pl.BlockSpec((BN, D), lambda i: (i, 0)),
        ],
        out_shape=[jax.ShapeDtypeStruct((N, D), _f32)] * 2,
    )(h, Wa, Wb)


def _sc_gather(row, col, A, B, coord16):
    mesh = plsc.VectorSubcoreMesh(core_axis_name="c", subcore_axis_name="s")

    @functools.partial(
        pl.kernel,
        mesh=mesh,
        out_type=[
            jax.ShapeDtypeStruct((E, D), _f32),
            jax.ShapeDtypeStruct((E, 16), _f32),
            jax.ShapeDtypeStruct((E, 16), _f32),
        ],
        scratch_types=[
            pltpu.VMEM((C,), _i32),
            pltpu.VMEM((C,), _i32),
            pltpu.VMEM((C, D), _f32),
            pltpu.VMEM((C, D), _f32),
            pltpu.VMEM((C, 16), _f32),
            pltpu.VMEM((C, 16), _f32),
            pltpu.SemaphoreType.DMA,
            pltpu.SemaphoreType.DMA,
            pltpu.SemaphoreType.DMA,
            pltpu.SemaphoreType.DMA,
        ],
        compiler_params=pltpu.CompilerParams(use_tc_tiling_on_sc=False),
    )
    def k(row_hbm, col_hbm, a_hbm, b_hbm, c16_hbm, s_hbm, xi_hbm, xk_hbm,
          ir_v, ic_v, ba_v, bb_v, xi_v, xk_v, s0, s1, s2, s3):
        wid = lax.axis_index("s") * NC + lax.axis_index("c")
        base = wid * PER_W

        def chunk(i, carry):
            off = base + i * C
            pltpu.sync_copy(row_hbm.at[pl.ds(off, C)], ir_v)
            pltpu.sync_copy(col_hbm.at[pl.ds(off, C)], ic_v)
            cp_a = pltpu.async_copy(a_hbm.at[ir_v], ba_v, s0)
            cp_b = pltpu.async_copy(b_hbm.at[ic_v], bb_v, s1)
            cp_xi = pltpu.async_copy(c16_hbm.at[ir_v], xi_v, s2)
            cp_xk = pltpu.async_copy(c16_hbm.at[ic_v], xk_v, s3)
            cp_xi.wait()
            cp_xk.wait()
            pltpu.sync_copy(xi_v, xi_hbm.at[pl.ds(off, C)])
            pltpu.sync_copy(xk_v, xk_hbm.at[pl.ds(off, C)])
            cp_a.wait()
            cp_b.wait()

            def rbody(r, rc):
                for j in range(D // 16):
                    sl2 = pl.ds(j * 16, 16)
                    ba_v[r, sl2] = ba_v[r, sl2] + bb_v[r, sl2]
                return rc

            lax.fori_loop(0, C, rbody, 0)
            pltpu.sync_copy(ba_v, s_hbm.at[pl.ds(off, C)])
            return carry

        lax.fori_loop(0, NCH, chunk, 0)

    return k(row, col, A, B, coord16)


def _edge_body(s_ref, xi_ref, xk_ref, rowm_ref, wg_ref, eb1_ref, ew2_ref,
               eb2_ref, f_ref, relb_ref):
    eps = 1e-8
    thr = 1e-6
    xi = xi_ref[...]
    xk = xk_ref[...]
    x0, x1, x2 = xi[:, 0:1], xi[:, 1:2], xi[:, 2:3]
    y0, y1, y2 = xk[:, 0:1], xk[:, 1:2], xk[:, 2:3]
    rx, ry, rz = x0 - y0, x1 - y1, x2 - y2
    radial = rx * rx + ry * ry + rz * rz
    rn = jnp.sqrt(radial) + eps
    ax, ay, az = rx / rn, ry / rn, rz / rn
    cx = x1 * y2 - x2 * y1
    cy = x2 * y0 - x0 * y2
    cz = x0 * y1 - x1 * y0
    cn = jnp.sqrt(cx * cx + cy * cy + cz * cz) + eps
    bx, by, bz = cx / cn, cy / cn, cz / cn
    ccx = ay * bz - az * by
    ccy = az * bx - ax * bz
    ccz = ax * by - ay * bx
    na = jnp.sqrt(ax * ax + ay * ay + az * az)
    nb = jnp.sqrt(bx * bx + by * by + bz * bz)
    nc = jnp.sqrt(ccx * ccx + ccy * ccy + ccz * ccz)
    mask = (na < thr) | (nb < thr) | (nc < thr)
    one = jnp.ones_like(ax)
    zero = jnp.zeros_like(ax)
    ax = jnp.where(mask, one, ax)
    ay = jnp.where(mask, zero, ay)
    az = jnp.where(mask, zero, az)
    bx = jnp.where(mask, zero, bx)
    by = jnp.where(mask, one, by)
    bz = jnp.where(mask, zero, bz)
    ccx = jnp.where(mask, zero, ccx)
    ccy = jnp.where(mask, zero, ccy)
    ccz = jnp.where(mask, one, ccz)
    zp = jnp.zeros((BE, 6), _f32)
    G = jnp.concatenate(
        [radial, ax, bx, ccx, ay, by, ccy, az, bz, ccz, zp], axis=1)
    pre = s_ref[...] + jnp.dot(G, wg_ref[...], preferred_element_type=_f32) \
        + eb1_ref[...]
    m = pre * jax.nn.sigmoid(pre)
    t = jnp.dot(m, ew2_ref[...], preferred_element_type=_f32) + eb2_ref[...]
    f_ref[...] = t * jax.nn.sigmoid(t)
    rel16 = jnp.concatenate([rx, ry, rz, one, jnp.zeros((BE, 12), _f32)],
                            axis=1)
    til = jnp.concatenate([rel16] * 8, axis=1)
    lg = jax.lax.broadcasted_iota(_i32, (BE, D), 1) // 16
    sel = lg == rowm_ref[...]
    relb_ref[...] = jnp.where(sel, til, jnp.zeros_like(til))


def _edge(S, xi16, xk16, rowm, Wg16, eb1, eW2, eb2):
    return pl.pallas_call(
        _edge_body,
        grid=(E // BE,),
        in_specs=[
            pl.BlockSpec((BE, D), lambda i: (i, 0)),
            pl.BlockSpec((BE, 16), lambda i: (i, 0)),
            pl.BlockSpec((BE, 16), lambda i: (i, 0)),
            pl.BlockSpec((BE, 1), lambda i: (i, 0)),
            pl.BlockSpec((16, D), lambda i: (0, 0)),
            pl.BlockSpec((1, D), lambda i: (0, 0)),
            pl.BlockSpec((D, D), lambda i: (0, 0)),
            pl.BlockSpec((1, D), lambda i: (0, 0)),
        ],
        out_specs=[
            pl.BlockSpec((BE, D), lambda i: (i, 0)),
            pl.BlockSpec((BE, D), lambda i: (i, 0)),
        ],
        out_shape=[
            jax.ShapeDtypeStruct((E, D), _f32),
            jax.ShapeDtypeStruct((E, D), _f32),
        ],
    )(S, xi16, xk16, rowm, Wg16, eb1, eW2, eb2)


def _sc_scatter(row, f, relB):
    mesh = plsc.VectorSubcoreMesh(core_axis_name="c", subcore_axis_name="s")

    @functools.partial(
        pl.kernel,
        mesh=mesh,
        out_type=[
            jax.ShapeDtypeStruct((NC * N, D), _f32),
            jax.ShapeDtypeStruct((NC * NCR, D), _f32),
        ],
        scratch_types=[
            pltpu.VMEM_SHARED((N, D), _f32),
            pltpu.VMEM_SHARED((NCR, D), _f32),
            pltpu.VMEM((C,), _i32),
            pltpu.VMEM((C,), _i32),
            pltpu.VMEM((C, D), _f32),
            pltpu.VMEM((C, D), _f32),
            pltpu.VMEM((125, D), _f32),
        ],
        compiler_params=pltpu.CompilerParams(use_tc_tiling_on_sc=False),
    )
    def k(row_hbm, f_hbm, relb_hbm, aggh_hbm, aggc_hbm,
          aggh_sh, aggc_sh, idx_v, idx2_v, f_v, rb_v, zb_v):
        cid = lax.axis_index("c")
        sid = lax.axis_index("s")
        rbase = sid * RPT
        cbase = sid * (NCR // NS)

        def zb_zero(r, rc):
            for j in range(D // 16):
                zb_v[r, pl.ds(j * 16, 16)] = jnp.zeros((16,), _f32)
            return rc

        lax.fori_loop(0, 125, zb_zero, 0)
        for kk in range(RPT // 125):
            pltpu.sync_copy(zb_v, aggh_sh.at[pl.ds(rbase + kk * 125, 125)])
        pltpu.sync_copy(zb_v.at[pl.ds(0, NCR // NS)],
                        aggc_sh.at[pl.ds(cbase, NCR // NS)])
        plsc.subcore_barrier()

        base = (cid * NS + sid) * PER_W

        def chunk(i, carry):
            off = base + i * C
            pltpu.sync_copy(row_hbm.at[pl.ds(off, C)], idx_v)
            for g in range(C // 16):
                sl = pl.ds(g * 16, 16)
                idx2_v[sl] = lax.shift_right_logical(idx_v[sl], 3)
            pltpu.sync_copy(f_hbm.at[pl.ds(off, C)], f_v)
            pltpu.sync_copy(relb_hbm.at[pl.ds(off, C)], rb_v)
            pltpu.sync_copy(f_v, aggh_sh.at[idx_v], add=True)
            pltpu.sync_copy(rb_v, aggc_sh.at[idx2_v], add=True)
            return carry

        lax.fori_loop(0, NCH, chunk, 0)
        plsc.subcore_barrier()

        obase = cid * N + rbase
        for kk in range(RPT // 125):
            pltpu.sync_copy(aggh_sh.at[pl.ds(rbase + kk * 125, 125)], zb_v)
            pltpu.sync_copy(zb_v, aggh_hbm.at[pl.ds(obase + kk * 125, 125)])
        pltpu.sync_copy(aggc_sh.at[pl.ds(cbase, NCR // NS)],
                        zb_v.at[pl.ds(0, NCR // NS)])
        pltpu.sync_copy(zb_v.at[pl.ds(0, NCR // NS)],
                        aggc_hbm.at[pl.ds(cid * NCR + cbase, NCR // NS)])

    return k(row, f, relB)


def _node_body(h_ref, c16_ref, aggh_ref, aggc_ref, w1a_ref, w1b_ref,
               nb1_ref, w2_ref, nb2_ref, ho_ref, co_ref):
    hb = h_ref[...]
    agg = aggh_ref[0] + aggh_ref[1]
    pre = (jnp.dot(hb, w1a_ref[...], preferred_element_type=_f32)
           + jnp.dot(agg, w1b_ref[...], preferred_element_type=_f32)
           + nb1_ref[...])
    hid = pre * jax.nn.sigmoid(pre)
    ho_ref[...] = hb + jnp.dot(hid, w2_ref[...], preferred_element_type=_f32) \
        + nb2_ref[...]
    s = aggc_ref[0] + aggc_ref[1]
    cnt = jnp.maximum(s[:, 3:4], 1.0)
    co_ref[...] = c16_ref[...] + s / cnt


def _node(h, coord16, aggH, aggC, nW1a, nW1b, nb1, nW2, nb2):
    return pl.pallas_call(
        _node_body,
        grid=(N // BN,),
        in_specs=[
            pl.BlockSpec((BN, D), lambda i: (i, 0)),
            pl.BlockSpec((BN, 16), lambda i: (i, 0)),
            pl.BlockSpec((NC, BN, D), lambda i: (0, i, 0)),
            pl.BlockSpec((NC, BN, 16), lambda i: (0, i, 0)),
            pl.BlockSpec((D, D), lambda i: (0, 0)),
            pl.BlockSpec((D, D), lambda i: (0, 0)),
            pl.BlockSpec((1, D), lambda i: (0, 0)),
            pl.BlockSpec((D, D), lambda i: (0, 0)),
            pl.BlockSpec((1, D), lambda i: (0, 0)),
        ],
        out_specs=[
            pl.BlockSpec((BN, D), lambda i: (i, 0)),
            pl.BlockSpec((BN, 16), lambda i: (i, 0)),
        ],
        out_shape=[
            jax.ShapeDtypeStruct((N, D), _f32),
            jax.ShapeDtypeStruct((N, 16), _f32),
        ],
    )(h, coord16, aggH, aggC, nW1a, nW1b, nb1, nW2, nb2)


def kernel(h, edge_index, coord, eW1, eb1, eW2, eb2, nW1, nb1, nW2, nb2):
    row = edge_index[0]
    col = edge_index[1]
    rowm = (row % 8).astype(_i32).reshape(E, 1)
    Wa = eW1[:D]
    Wb = eW1[D:2 * D]
    Wg16 = jnp.pad(eW1[2 * D:], ((0, 6), (0, 0)))
    coord16 = jnp.pad(coord, ((0, 0), (0, 13)))
    A, B = _precompute(h, Wa, Wb)
    S, xi16, xk16 = _sc_gather(row, col, A, B, coord16)
    f, relB = _edge(S, xi16, xk16, rowm, Wg16,
                    eb1.reshape(1, D), eW2, eb2.reshape(1, D))
    aggH, aggC2 = _sc_scatter(row, f, relB)
    aggC = aggC2.reshape(NC, NCR * 8, 16)[:, :N]
    h_out, co16 = _node(h, coord16, aggH.reshape(NC, N, D), aggC,
                        nW1[:D], nW1[D:], nb1.reshape(1, D), nW2,
                        nb2.reshape(1, D))
    return h_out, co16[:, :3]


# speedup vs baseline: 4.2984x; 2.2277x over previous
import functools

import jax
import jax.numpy as jnp
from jax import lax
from jax.experimental import pallas as pl
from jax.experimental.pallas import tpu as pltpu
from jax.experimental.pallas import tpu_sc as plsc

N = 10000
E = 320000
D = 128
NC, NS = 2, 16
NW = NC * NS
PER_W = E // NW
C = 80
NCH = PER_W // C
RPT = N // NS
BN = 400
BE = 512

_f32 = jnp.float32
_i32 = jnp.int32


def _pre_body(h_ref, wa_ref, wb_ref, a_ref, b_ref):
    hb = h_ref[...]
    a_ref[...] = jnp.dot(hb, wa_ref[...], preferred_element_type=_f32)
    b_ref[...] = jnp.dot(hb, wb_ref[...], preferred_element_type=_f32)


def _precompute(h, Wa, Wb):
    return pl.pallas_call(
        _pre_body,
        grid=(N // BN,),
        in_specs=[
            pl.BlockSpec((BN, D), lambda i: (i, 0)),
            pl.BlockSpec((D, D), lambda i: (0, 0)),
            pl.BlockSpec((D, D), lambda i: (0, 0)),
        ],
        out_specs=[
            pl.BlockSpec((BN, D), lambda i: (i, 0)),
            pl.BlockSpec((BN, D), lambda i: (i, 0)),
        ],
        out_shape=[jax.ShapeDtypeStruct((N, D), _f32)] * 2,
    )(h, Wa, Wb)


def _sc_gather(row, col, A, B, coord16):
    mesh = plsc.VectorSubcoreMesh(core_axis_name="c", subcore_axis_name="s")

    @functools.partial(
        pl.kernel,
        mesh=mesh,
        out_type=[
            jax.ShapeDtypeStruct((E, D), _f32),
            jax.ShapeDtypeStruct((E, 16), _f32),
            jax.ShapeDtypeStruct((E, 16), _f32),
        ],
        scratch_types=[
            pltpu.VMEM((C,), _i32),
            pltpu.VMEM((C,), _i32),
            pltpu.VMEM((C, D), _f32),
            pltpu.VMEM((C, D), _f32),
            pltpu.VMEM((C, 16), _f32),
            pltpu.VMEM((C, 16), _f32),
            pltpu.SemaphoreType.DMA,
            pltpu.SemaphoreType.DMA,
            pltpu.SemaphoreType.DMA,
            pltpu.SemaphoreType.DMA,
        ],
        compiler_params=pltpu.CompilerParams(use_tc_tiling_on_sc=False),
    )
    def k(row_hbm, col_hbm, a_hbm, b_hbm, c16_hbm, s_hbm, xi_hbm, xk_hbm,
          ir_v, ic_v, ba_v, bb_v, xi_v, xk_v, s0, s1, s2, s3):
        wid = lax.axis_index("s") * NC + lax.axis_index("c")
        base = wid * PER_W

        def chunk(i, carry):
            off = base + i * C
            pltpu.sync_copy(row_hbm.at[pl.ds(off, C)], ir_v)
            pltpu.sync_copy(col_hbm.at[pl.ds(off, C)], ic_v)
            cp_a = pltpu.async_copy(a_hbm.at[ir_v], ba_v, s0)
            cp_b = pltpu.async_copy(b_hbm.at[ic_v], bb_v, s1)
            cp_xi = pltpu.async_copy(c16_hbm.at[ir_v], xi_v, s2)
            cp_xk = pltpu.async_copy(c16_hbm.at[ic_v], xk_v, s3)
            cp_xi.wait()
            cp_xk.wait()
            pltpu.sync_copy(xi_v, xi_hbm.at[pl.ds(off, C)])
            pltpu.sync_copy(xk_v, xk_hbm.at[pl.ds(off, C)])
            cp_a.wait()
            cp_b.wait()

            def rbody(r, rc):
                for j in range(D // 16):
                    sl2 = pl.ds(j * 16, 16)
                    ba_v[r, sl2] = ba_v[r, sl2] + bb_v[r, sl2]
                return rc

            lax.fori_loop(0, C, rbody, 0)
            pltpu.sync_copy(ba_v, s_hbm.at[pl.ds(off, C)])
            return carry

        lax.fori_loop(0, NCH, chunk, 0)

    return k(row, col, A, B, coord16)


def _roll(x, k):
    return jnp.roll(x, k, axis=1)


BG = 800


def _geom_body(xp_ref, kp_ref, gp_ref, r16_ref):
    eps = 1e-8
    thr = 1e-6
    xi = xp_ref[...]
    xk = kp_ref[...]
    li = jax.lax.broadcasted_iota(_i32, (1, D), 1) % 16
    m012 = (li < 3).astype(_f32)
    ml0 = (li == 0).astype(_f32)
    ea = ml0
    eb_ = (li == 1).astype(_f32)
    ec = (li == 2).astype(_f32)
    e3 = (li == 3).astype(_f32)
    rel = xi - xk
    cp = (_roll(xi, -1) * _roll(xk, -2) - _roll(xi, -2) * _roll(xk, -1))
    cpm = cp * m012
    cpe = cpm + _roll(cpm, 3)
    cc = (_roll(rel, -1) * _roll(cpe, -2) - _roll(rel, -2) * _roll(cpe, -1))
    ccm = cc * m012
    cce = ccm + _roll(ccm, 3)
    rel2 = rel * rel
    radial = rel2 + _roll(rel2, -1) + _roll(rel2, -2)
    cpe2 = cpe * cpe
    cn2 = cpe2 + _roll(cpe2, -1) + _roll(cpe2, -2)
    cce2 = cce * cce
    nc2 = cce2 + _roll(cce2, -1) + _roll(cce2, -2)
    rn = jnp.sqrt(radial)
    cn = jnp.sqrt(cn2)
    inv_r = 1.0 / (rn + eps)
    inv_c = 1.0 / (cn + eps)
    na = rn * inv_r
    nb = cn * inv_c
    irc = inv_r * inv_c
    ncn = jnp.sqrt(nc2) * irc
    mask = (na < thr) | (nb < thr) | (ncn < thr)
    a_m = jnp.where(mask, ea, rel * inv_r)
    b_m = jnp.where(mask, eb_, cpe * inv_c)
    c_m = jnp.where(mask, ec, cce * irc)
    gp_ref[...] = (radial * ml0 + _roll(a_m * m012, 1)
                   + _roll(b_m * m012, 4) + _roll(c_m * m012, 7))
    r16_ref[...] = rel * m012 + e3


def _geom(xip, xkp):
    return pl.pallas_call(
        _geom_body,
        grid=(E // 8 // BG,),
        in_specs=[
            pl.BlockSpec((BG, D), lambda i: (i, 0)),
            pl.BlockSpec((BG, D), lambda i: (i, 0)),
        ],
        out_specs=[
            pl.BlockSpec((BG, D), lambda i: (i, 0)),
            pl.BlockSpec((BG, D), lambda i: (i, 0)),
        ],
        out_shape=[
            jax.ShapeDtypeStruct((E // 8, D), _f32),
            jax.ShapeDtypeStruct((E // 8, D), _f32),
        ],
    )(xip, xkp)


def _mlp_body(s_ref, g_ref, wgo_ref, eb1_ref, ew2_ref, eb2_ref, f_ref):
    geo = jnp.dot(g_ref[...], wgo_ref[...], preferred_element_type=_f32)
    pre = s_ref[...] + geo + eb1_ref[...]
    m = pre * jax.nn.sigmoid(pre)
    t = jnp.dot(m, ew2_ref[...], preferred_element_type=_f32) + eb2_ref[...]
    f_ref[...] = t * jax.nn.sigmoid(t)


def _mlp(S, G, WgOrd, eb1, eW2, eb2):
    return pl.pallas_call(
        _mlp_body,
        grid=(E // BE,),
        in_specs=[
            pl.BlockSpec((BE, D), lambda i: (i, 0)),
            pl.BlockSpec((BE, 16), lambda i: (i, 0)),
            pl.BlockSpec((16, D), lambda i: (0, 0)),
            pl.BlockSpec((1, D), lambda i: (0, 0)),
            pl.BlockSpec((D, D), lambda i: (0, 0)),
            pl.BlockSpec((1, D), lambda i: (0, 0)),
        ],
        out_specs=[pl.BlockSpec((BE, D), lambda i: (i, 0))],
        out_shape=[jax.ShapeDtypeStruct((E, D), _f32)],
    )(S, G, WgOrd, eb1, eW2, eb2)


def _sc_scatter(row, f, rel16):
    mesh = plsc.VectorSubcoreMesh(core_axis_name="c", subcore_axis_name="s")

    @functools.partial(
        pl.kernel,
        mesh=mesh,
        out_type=[
            jax.ShapeDtypeStruct((NC * N, D), _f32),
            jax.ShapeDtypeStruct((NC * N, 16), _f32),
        ],
        scratch_types=[
            pltpu.VMEM_SHARED((N, D), _f32),
            pltpu.VMEM_SHARED((N, 16), _f32),
            pltpu.VMEM((C,), _i32),
            pltpu.VMEM((C, D), _f32),
            pltpu.VMEM((C, 16), _f32),
            pltpu.VMEM((125, D), _f32),
            pltpu.VMEM((RPT, 16), _f32),
        ],
        compiler_params=pltpu.CompilerParams(use_tc_tiling_on_sc=False),
    )
    def k(row_hbm, f_hbm, rel_hbm, aggh_hbm, aggc_hbm,
          aggh_sh, aggc_sh, idx_v, f_v, rel_v, zb_v, zb16_v):
        cid = lax.axis_index("c")
        sid = lax.axis_index("s")
        rbase = sid * RPT

        def zb_zero(r, rc):
            for j in range(D // 16):
                zb_v[r, pl.ds(j * 16, 16)] = jnp.zeros((16,), _f32)
            return rc

        lax.fori_loop(0, 125, zb_zero, 0)

        def zb16_zero(r, rc):
            zb16_v[r, pl.ds(0, 16)] = jnp.zeros((16,), _f32)
            return rc

        lax.fori_loop(0, RPT, zb16_zero, 0)
        for kk in range(RPT // 125):
            pltpu.sync_copy(zb_v, aggh_sh.at[pl.ds(rbase + kk * 125, 125)])
        pltpu.sync_copy(zb16_v, aggc_sh.at[pl.ds(rbase, RPT)])
        plsc.subcore_barrier()

        base = (cid * NS + sid) * PER_W

        def chunk(i, carry):
            off = base + i * C
            pltpu.sync_copy(row_hbm.at[pl.ds(off, C)], idx_v)
            pltpu.sync_copy(f_hbm.at[pl.ds(off, C)], f_v)
            pltpu.sync_copy(rel_hbm.at[pl.ds(off, C)], rel_v)
            pltpu.sync_copy(f_v, aggh_sh.at[idx_v], add=True)
            pltpu.sync_copy(rel_v, aggc_sh.at[idx_v], add=True)
            return carry

        lax.fori_loop(0, NCH, chunk, 0)
        plsc.subcore_barrier()

        obase = cid * N + rbase
        for kk in range(RPT // 125):
            pltpu.sync_copy(aggh_sh.at[pl.ds(rbase + kk * 125, 125)], zb_v)
            pltpu.sync_copy(zb_v, aggh_hbm.at[pl.ds(obase + kk * 125, 125)])
        pltpu.sync_copy(aggc_sh.at[pl.ds(rbase, RPT)], zb16_v)
        pltpu.sync_copy(zb16_v, aggc_hbm.at[pl.ds(obase, RPT)])

    return k(row, f, rel16)


def _node_body(h_ref, c16_ref, aggh_ref, aggc_ref, w1a_ref, w1b_ref,
               nb1_ref, w2_ref, nb2_ref, ho_ref, co_ref):
    hb = h_ref[...]
    agg = aggh_ref[0] + aggh_ref[1]
    pre = (jnp.dot(hb, w1a_ref[...], preferred_element_type=_f32)
           + jnp.dot(agg, w1b_ref[...], preferred_element_type=_f32)
           + nb1_ref[...])
    hid = pre * jax.nn.sigmoid(pre)
    ho_ref[...] = hb + jnp.dot(hid, w2_ref[...], preferred_element_type=_f32) \
        + nb2_ref[...]
    s = aggc_ref[0] + aggc_ref[1]
    cnt = jnp.maximum(s[:, 3:4], 1.0)
    co_ref[...] = c16_ref[...] + s / cnt


def _node(h, coord16, aggH, aggC, nW1a, nW1b, nb1, nW2, nb2):
    return pl.pallas_call(
        _node_body,
        grid=(N // BN,),
        in_specs=[
            pl.BlockSpec((BN, D), lambda i: (i, 0)),
            pl.BlockSpec((BN, 16), lambda i: (i, 0)),
            pl.BlockSpec((NC, BN, D), lambda i: (0, i, 0)),
            pl.BlockSpec((NC, BN, 16), lambda i: (0, i, 0)),
            pl.BlockSpec((D, D), lambda i: (0, 0)),
            pl.BlockSpec((D, D), lambda i: (0, 0)),
            pl.BlockSpec((1, D), lambda i: (0, 0)),
            pl.BlockSpec((D, D), lambda i: (0, 0)),
            pl.BlockSpec((1, D), lambda i: (0, 0)),
        ],
        out_specs=[
            pl.BlockSpec((BN, D), lambda i: (i, 0)),
            pl.BlockSpec((BN, 16), lambda i: (i, 0)),
        ],
        out_shape=[
            jax.ShapeDtypeStruct((N, D), _f32),
            jax.ShapeDtypeStruct((N, 16), _f32),
        ],
    )(h, coord16, aggH, aggC, nW1a, nW1b, nb1, nW2, nb2)


def kernel(h, edge_index, coord, eW1, eb1, eW2, eb2, nW1, nb1, nW2, nb2):
    row = edge_index[0]
    col = edge_index[1]
    Wa = eW1[:D]
    Wb = eW1[D:2 * D]
    Wg = eW1[2 * D:]
    WgOrd = jnp.concatenate(
        [Wg[0:1], Wg[1:2], Wg[4:5], Wg[7:8], Wg[2:3], Wg[5:6], Wg[8:9],
         Wg[3:4], Wg[6:7], Wg[9:10], jnp.zeros((6, D), _f32)], 0)
    coord16 = jnp.concatenate([coord] * 5 + [jnp.zeros((N, 1), _f32)], 1)
    A, B = _precompute(h, Wa, Wb)
    S, xi16, xk16 = _sc_gather(row, col, A, B, coord16)
    Gp, rel16p = _geom(xi16.reshape(E // 8, D), xk16.reshape(E // 8, D))
    (f,) = _mlp(S, Gp.reshape(E, 16), WgOrd,
                eb1.reshape(1, D), eW2, eb2.reshape(1, D))
    aggH, aggC = _sc_scatter(row, f, rel16p.reshape(E, 16))
    aggC = aggC.reshape(NC, N, 16)
    h_out, co16 = _node(h, coord16, aggH.reshape(NC, N, D), aggC,
                        nW1[:D], nW1[D:], nb1.reshape(1, D), nW2,
                        nb2.reshape(1, D))
    return h_out, co16[:, :3]


# speedup vs baseline: 5.9756x; 1.3902x over previous
import functools

import jax
import jax.numpy as jnp
from jax import lax
from jax.experimental import pallas as pl
from jax.experimental.pallas import tpu as pltpu
from jax.experimental.pallas import tpu_sc as plsc

N = 10000
E = 320000
D = 128
NC, NS = 2, 16
NW = NC * NS
PER_W = E // NW
C = 80
NCH = PER_W // C
RPT = N // NS
BN = 400
BE = 512

_f32 = jnp.float32
_i32 = jnp.int32


def _pre_body(h_ref, wa_ref, wb_ref, a_ref, b_ref):
    hb = h_ref[...]
    a_ref[...] = jnp.dot(hb, wa_ref[...], preferred_element_type=_f32)
    b_ref[...] = jnp.dot(hb, wb_ref[...], preferred_element_type=_f32)


def _precompute(h, Wa, Wb):
    return pl.pallas_call(
        _pre_body,
        grid=(N // BN,),
        in_specs=[
            pl.BlockSpec((BN, D), lambda i: (i, 0)),
            pl.BlockSpec((D, D), lambda i: (0, 0)),
            pl.BlockSpec((D, D), lambda i: (0, 0)),
        ],
        out_specs=[
            pl.BlockSpec((BN, D), lambda i: (i, 0)),
            pl.BlockSpec((BN, D), lambda i: (i, 0)),
        ],
        out_shape=[jax.ShapeDtypeStruct((N, D), _f32)] * 2,
    )(h, Wa, Wb)


def _sc_gather(row, col, A, B, coord16):
    mesh = plsc.VectorSubcoreMesh(core_axis_name="c", subcore_axis_name="s")

    @functools.partial(
        pl.kernel,
        mesh=mesh,
        out_type=[
            jax.ShapeDtypeStruct((E, D), _f32),
            jax.ShapeDtypeStruct((E, 16), _f32),
            jax.ShapeDtypeStruct((E, 16), _f32),
        ],
        scratch_types=[
            pltpu.VMEM((C,), _i32), pltpu.VMEM((C,), _i32),
            pltpu.VMEM((C,), _i32), pltpu.VMEM((C,), _i32),
            pltpu.VMEM((C, D), _f32), pltpu.VMEM((C, D), _f32),
            pltpu.VMEM((C, D), _f32), pltpu.VMEM((C, D), _f32),
            pltpu.VMEM((C, D), _f32), pltpu.VMEM((C, D), _f32),
            pltpu.VMEM((C, 16), _f32), pltpu.VMEM((C, 16), _f32),
            pltpu.VMEM((C, 16), _f32), pltpu.VMEM((C, 16), _f32),
        ] + [pltpu.SemaphoreType.DMA] * 10,
        compiler_params=pltpu.CompilerParams(use_tc_tiling_on_sc=False),
    )
    def k(row_hbm, col_hbm, a_hbm, b_hbm, c16_hbm, s_hbm, xi_hbm, xk_hbm,
          ir0, ir1, ic0, ic1, ba0, ba1, bb0, bb1, sb0, sb1, xi0, xi1,
          xk0, xk1, si0, si1, sg0, sg1, sxo0, sxo1, sko0, sko1, ss0, ss1):
        wid = lax.axis_index("s") * NC + lax.axis_index("c")
        base = wid * PER_W
        irs, ics = (ir0, ir1), (ic0, ic1)
        bas, bbs, sbs = (ba0, ba1), (bb0, bb1), (sb0, sb1)
        xis, xks = (xi0, xi1), (xk0, xk1)
        sis, sgs = (si0, si1), (sg0, sg1)
        sxos, skos, sss = (sxo0, sxo1), (sko0, sko1), (ss0, ss1)

        def issue_idx(i, p):
            off = base + i * C
            pltpu.async_copy(row_hbm.at[pl.ds(off, C)], irs[p], sis[p])
            pltpu.async_copy(col_hbm.at[pl.ds(off, C)], ics[p], sis[p])

        def wait_idx(p):
            pltpu.make_async_copy(row_hbm.at[pl.ds(0, C)], irs[p],
                                  sis[p]).wait()
            pltpu.make_async_copy(col_hbm.at[pl.ds(0, C)], ics[p],
                                  sis[p]).wait()

        def _maybe(cond, fn):
            if isinstance(cond, bool):
                if cond:
                    fn()
            else:
                pl.when(cond)(fn)

        def issue_g(i, p):
            def drain():
                pltpu.make_async_copy(xis[p], xi_hbm.at[pl.ds(0, C)],
                                      sxos[p]).wait()
                pltpu.make_async_copy(xks[p], xk_hbm.at[pl.ds(0, C)],
                                      skos[p]).wait()

            _maybe(i >= 2, drain)
            pltpu.async_copy(a_hbm.at[irs[p]], bas[p], sgs[p])
            pltpu.async_copy(b_hbm.at[ics[p]], bbs[p], sgs[p])
            pltpu.async_copy(c16_hbm.at[irs[p]], xis[p], sgs[p])
            pltpu.async_copy(c16_hbm.at[ics[p]], xks[p], sgs[p])

        def process(i, p):
            off = base + i * C
            pltpu.make_async_copy(a_hbm.at[pl.ds(0, C)], bas[p],
                                  sgs[p]).wait()
            pltpu.make_async_copy(b_hbm.at[pl.ds(0, C)], bbs[p],
                                  sgs[p]).wait()
            pltpu.make_async_copy(c16_hbm.at[pl.ds(0, C)], xis[p],
                                  sgs[p]).wait()
            pltpu.make_async_copy(c16_hbm.at[pl.ds(0, C)], xks[p],
                                  sgs[p]).wait()
            pltpu.async_copy(xis[p], xi_hbm.at[pl.ds(off, C)], sxos[p])
            pltpu.async_copy(xks[p], xk_hbm.at[pl.ds(off, C)], skos[p])

            def drain_s():
                pltpu.make_async_copy(sbs[p], s_hbm.at[pl.ds(0, C)],
                                      sss[p]).wait()

            _maybe(i >= 2, drain_s)
            ba_v, bb_v, sb_v = bas[p], bbs[p], sbs[p]

            def rbody(r, rc):
                for j in range(D // 16):
                    sl2 = pl.ds(j * 16, 16)
                    sb_v[r, sl2] = ba_v[r, sl2] + bb_v[r, sl2]
                return rc

            lax.fori_loop(0, C, rbody, 0)
            pltpu.async_copy(sbs[p], s_hbm.at[pl.ds(off, C)], sss[p])

        issue_idx(0, 0)
        wait_idx(0)
        issue_g(0, 0)
        issue_idx(1, 1)

        def body(j, carry):
            wait_idx(1)
            issue_g(2 * j + 1, 1)
            process(2 * j, 0)
            issue_idx(2 * j + 2, 0)
            wait_idx(0)
            issue_g(2 * j + 2, 0)
            process(2 * j + 1, 1)

            @pl.when(j < NCH // 2 - 1)
            def _():
                issue_idx(2 * j + 3, 1)

            return carry

        lax.fori_loop(0, NCH // 2, body, 0)
        process(NCH - 1, 0)
        for p in range(2):
            pltpu.make_async_copy(xis[p], xi_hbm.at[pl.ds(0, C)],
                                  sxos[p]).wait()
            pltpu.make_async_copy(xks[p], xk_hbm.at[pl.ds(0, C)],
                                  skos[p]).wait()
            pltpu.make_async_copy(sbs[p], s_hbm.at[pl.ds(0, C)],
                                  sss[p]).wait()

    return k(row, col, A, B, coord16)


def _roll(x, k):
    return jnp.roll(x, k, axis=1)


BG = 800


def _geom_body(xp_ref, kp_ref, gp_ref, r16_ref):
    eps = 1e-8
    thr = 1e-6
    xi = xp_ref[...]
    xk = kp_ref[...]
    li = jax.lax.broadcasted_iota(_i32, (1, D), 1) % 16
    m012 = (li < 3).astype(_f32)
    ml0 = (li == 0).astype(_f32)
    ea = ml0
    eb_ = (li == 1).astype(_f32)
    ec = (li == 2).astype(_f32)
    e3 = (li == 3).astype(_f32)
    rel = xi - xk
    cp = (_roll(xi, -1) * _roll(xk, -2) - _roll(xi, -2) * _roll(xk, -1))
    cpm = cp * m012
    cpe = cpm + _roll(cpm, 3)
    cc = (_roll(rel, -1) * _roll(cpe, -2) - _roll(rel, -2) * _roll(cpe, -1))
    ccm = cc * m012
    cce = ccm + _roll(ccm, 3)
    rel2 = rel * rel
    radial = rel2 + _roll(rel2, -1) + _roll(rel2, -2)
    cpe2 = cpe * cpe
    cn2 = cpe2 + _roll(cpe2, -1) + _roll(cpe2, -2)
    cce2 = cce * cce
    nc2 = cce2 + _roll(cce2, -1) + _roll(cce2, -2)
    rn = jnp.sqrt(radial)
    cn = jnp.sqrt(cn2)
    inv_r = 1.0 / (rn + eps)
    inv_c = 1.0 / (cn + eps)
    na = rn * inv_r
    nb = cn * inv_c
    irc = inv_r * inv_c
    ncn = jnp.sqrt(nc2) * irc
    mask = (na < thr) | (nb < thr) | (ncn < thr)
    a_m = jnp.where(mask, ea, rel * inv_r)
    b_m = jnp.where(mask, eb_, cpe * inv_c)
    c_m = jnp.where(mask, ec, cce * irc)
    gp_ref[...] = (radial * ml0 + _roll(a_m * m012, 1)
                   + _roll(b_m * m012, 4) + _roll(c_m * m012, 7))
    r16_ref[...] = rel * m012 + e3


def _geom(xip, xkp):
    return pl.pallas_call(
        _geom_body,
        grid=(E // 8 // BG,),
        in_specs=[
            pl.BlockSpec((BG, D), lambda i: (i, 0)),
            pl.BlockSpec((BG, D), lambda i: (i, 0)),
        ],
        out_specs=[
            pl.BlockSpec((BG, D), lambda i: (i, 0)),
            pl.BlockSpec((BG, D), lambda i: (i, 0)),
        ],
        out_shape=[
            jax.ShapeDtypeStruct((E // 8, D), _f32),
            jax.ShapeDtypeStruct((E // 8, D), _f32),
        ],
    )(xip, xkp)


def _mlp_body(s_ref, g_ref, wgo_ref, eb1_ref, ew2_ref, eb2_ref, f_ref):
    geo = jnp.dot(g_ref[...], wgo_ref[...], preferred_element_type=_f32)
    pre = s_ref[...] + geo + eb1_ref[...]
    m = pre * jax.nn.sigmoid(pre)
    t = jnp.dot(m, ew2_ref[...], preferred_element_type=_f32) + eb2_ref[...]
    f_ref[...] = t * jax.nn.sigmoid(t)


def _mlp(S, G, WgOrd, eb1, eW2, eb2):
    return pl.pallas_call(
        _mlp_body,
        grid=(E // BE,),
        in_specs=[
            pl.BlockSpec((BE, D), lambda i: (i, 0)),
            pl.BlockSpec((BE, 16), lambda i: (i, 0)),
            pl.BlockSpec((16, D), lambda i: (0, 0)),
            pl.BlockSpec((1, D), lambda i: (0, 0)),
            pl.BlockSpec((D, D), lambda i: (0, 0)),
            pl.BlockSpec((1, D), lambda i: (0, 0)),
        ],
        out_specs=[pl.BlockSpec((BE, D), lambda i: (i, 0))],
        out_shape=[jax.ShapeDtypeStruct((E, D), _f32)],
    )(S, G, WgOrd, eb1, eW2, eb2)


def _sc_scatter(row, f, rel16):
    mesh = plsc.VectorSubcoreMesh(core_axis_name="c", subcore_axis_name="s")

    @functools.partial(
        pl.kernel,
        mesh=mesh,
        out_type=[
            jax.ShapeDtypeStruct((NC * N, D), _f32),
            jax.ShapeDtypeStruct((NC * N, 16), _f32),
        ],
        scratch_types=[
            pltpu.VMEM_SHARED((N, D), _f32),
            pltpu.VMEM_SHARED((N, 16), _f32),
            pltpu.VMEM((C,), _i32), pltpu.VMEM((C,), _i32),
            pltpu.VMEM((C, D), _f32), pltpu.VMEM((C, D), _f32),
            pltpu.VMEM((C, 16), _f32), pltpu.VMEM((C, 16), _f32),
            pltpu.SemaphoreType.DMA,
            pltpu.SemaphoreType.DMA,
        ],
        compiler_params=pltpu.CompilerParams(use_tc_tiling_on_sc=False),
    )
    def k(row_hbm, f_hbm, rel_hbm, aggh_hbm, aggc_hbm,
          aggh_sh, aggc_sh, ix0, ix1, f0, f1, r0, r1, sl0, sl1):
        cid = lax.axis_index("c")
        sid = lax.axis_index("s")
        rbase = sid * RPT
        ixs, fs, rs, sls = (ix0, ix1), (f0, f1), (r0, r1), (sl0, sl1)
        _chunks = [(kk * C, C) for kk in range(RPT // C)] + \
            [(RPT - RPT % C, RPT % C)]

        def f0_zero(r, rc):
            for j in range(D // 16):
                f0[r, pl.ds(j * 16, 16)] = jnp.zeros((16,), _f32)
            return rc

        lax.fori_loop(0, C, f0_zero, 0)

        def r0_zero(r, rc):
            r0[r, pl.ds(0, 16)] = jnp.zeros((16,), _f32)
            return rc

        lax.fori_loop(0, C, r0_zero, 0)
        for (o, ln) in _chunks:
            pltpu.sync_copy(f0.at[pl.ds(0, ln)],
                            aggh_sh.at[pl.ds(rbase + o, ln)])
            pltpu.sync_copy(r0.at[pl.ds(0, ln)],
                            aggc_sh.at[pl.ds(rbase + o, ln)])
        plsc.subcore_barrier()

        base = (cid * NS + sid) * PER_W

        def issue_loads(i, p):
            off = base + i * C
            pltpu.async_copy(row_hbm.at[pl.ds(off, C)], ixs[p], sls[p])
            pltpu.async_copy(f_hbm.at[pl.ds(off, C)], fs[p], sls[p])
            pltpu.async_copy(rel_hbm.at[pl.ds(off, C)], rs[p], sls[p])

        def scatter(p):
            pltpu.make_async_copy(row_hbm.at[pl.ds(0, C)], ixs[p],
                                  sls[p]).wait()
            pltpu.make_async_copy(f_hbm.at[pl.ds(0, C)], fs[p],
                                  sls[p]).wait()
            pltpu.make_async_copy(rel_hbm.at[pl.ds(0, C)], rs[p],
                                  sls[p]).wait()
            pltpu.sync_copy(fs[p], aggh_sh.at[ixs[p]], add=True)
            pltpu.sync_copy(rs[p], aggc_sh.at[ixs[p]], add=True)

        issue_loads(0, 0)
        issue_loads(1, 1)

        def body(j, carry):
            scatter(0)
            issue_loads(2 * j + 2, 0)
            scatter(1)

            @pl.when(j < NCH // 2 - 1)
            def _():
                issue_loads(2 * j + 3, 1)

            return carry

        lax.fori_loop(0, NCH // 2, body, 0)
        scatter(0)
        plsc.subcore_barrier()

        obase = cid * N + rbase
        for (o, ln) in _chunks:
            pltpu.sync_copy(aggh_sh.at[pl.ds(rbase + o, ln)],
                            f0.at[pl.ds(0, ln)])
            pltpu.sync_copy(f0.at[pl.ds(0, ln)],
                            aggh_hbm.at[pl.ds(obase + o, ln)])
            pltpu.sync_copy(aggc_sh.at[pl.ds(rbase + o, ln)],
                            r0.at[pl.ds(0, ln)])
            pltpu.sync_copy(r0.at[pl.ds(0, ln)],
                            aggc_hbm.at[pl.ds(obase + o, ln)])

    return k(row, f, rel16)


def _node_body(h_ref, c16_ref, aggh_ref, aggc_ref, w1a_ref, w1b_ref,
               nb1_ref, w2_ref, nb2_ref, ho_ref, co_ref):
    hb = h_ref[...]
    agg = aggh_ref[0] + aggh_ref[1]
    pre = (jnp.dot(hb, w1a_ref[...], preferred_element_type=_f32)
           + jnp.dot(agg, w1b_ref[...], preferred_element_type=_f32)
           + nb1_ref[...])
    hid = pre * jax.nn.sigmoid(pre)
    ho_ref[...] = hb + jnp.dot(hid, w2_ref[...], preferred_element_type=_f32) \
        + nb2_ref[...]
    s = aggc_ref[0] + aggc_ref[1]
    cnt = jnp.maximum(s[:, 3:4], 1.0)
    co_ref[...] = c16_ref[...] + s / cnt


def _node(h, coord16, aggH, aggC, nW1a, nW1b, nb1, nW2, nb2):
    return pl.pallas_call(
        _node_body,
        grid=(N // BN,),
        in_specs=[
            pl.BlockSpec((BN, D), lambda i: (i, 0)),
            pl.BlockSpec((BN, 16), lambda i: (i, 0)),
            pl.BlockSpec((NC, BN, D), lambda i: (0, i, 0)),
            pl.BlockSpec((NC, BN, 16), lambda i: (0, i, 0)),
            pl.BlockSpec((D, D), lambda i: (0, 0)),
            pl.BlockSpec((D, D), lambda i: (0, 0)),
            pl.BlockSpec((1, D), lambda i: (0, 0)),
            pl.BlockSpec((D, D), lambda i: (0, 0)),
            pl.BlockSpec((1, D), lambda i: (0, 0)),
        ],
        out_specs=[
            pl.BlockSpec((BN, D), lambda i: (i, 0)),
            pl.BlockSpec((BN, 16), lambda i: (i, 0)),
        ],
        out_shape=[
            jax.ShapeDtypeStruct((N, D), _f32),
            jax.ShapeDtypeStruct((N, 16), _f32),
        ],
    )(h, coord16, aggH, aggC, nW1a, nW1b, nb1, nW2, nb2)


def kernel(h, edge_index, coord, eW1, eb1, eW2, eb2, nW1, nb1, nW2, nb2):
    row = edge_index[0]
    col = edge_index[1]
    Wa = eW1[:D]
    Wb = eW1[D:2 * D]
    Wg = eW1[2 * D:]
    WgOrd = jnp.concatenate(
        [Wg[0:1], Wg[1:2], Wg[4:5], Wg[7:8], Wg[2:3], Wg[5:6], Wg[8:9],
         Wg[3:4], Wg[6:7], Wg[9:10], jnp.zeros((6, D), _f32)], 0)
    coord16 = jnp.concatenate([coord] * 5 + [jnp.zeros((N, 1), _f32)], 1)
    A, B = _precompute(h, Wa, Wb)
    S, xi16, xk16 = _sc_gather(row, col, A, B, coord16)
    Gp, rel16p = _geom(xi16.reshape(E // 8, D), xk16.reshape(E // 8, D))
    (f,) = _mlp(S, Gp.reshape(E, 16), WgOrd,
                eb1.reshape(1, D), eW2, eb2.reshape(1, D))
    aggH, aggC = _sc_scatter(row, f, rel16p.reshape(E, 16))
    aggC = aggC.reshape(NC, N, 16)
    h_out, co16 = _node(h, coord16, aggH.reshape(NC, N, D), aggC,
                        nW1[:D], nW1[D:], nb1.reshape(1, D), nW2,
                        nb2.reshape(1, D))
    return h_out, co16[:, :3]
